# baseline (speedup 1.0000x reference)
import functools

import jax
import jax.numpy as jnp
from jax import lax
from jax.experimental import pallas as pl
from jax.experimental.pallas import tpu as pltpu
from jax.experimental.pallas import tpu_sc as plsc

_D = 32
_CHUNK = 1280
_SPLIT = 524288
_IGN = -1
_NL = 16


def _make_lookup(total, num_workers, num_cores):
    bpw = total // num_workers
    nsup = bpw // _CHUNK
    mesh = plsc.VectorSubcoreMesh(core_axis_name="c", subcore_axis_name="s")

    @functools.partial(
        pl.kernel,
        out_type=jax.ShapeDtypeStruct((total, _D), jnp.float32),
        mesh=mesh,
        scratch_types=[
            pltpu.VMEM((nsup, _CHUNK), jnp.int32),
            pltpu.VMEM((nsup, _CHUNK), jnp.int32),
            pltpu.VMEM((nsup, _CHUNK), jnp.int32),
            pltpu.VMEM((2, _CHUNK, _D), jnp.float32),
            pltpu.SemaphoreType.DMA,
            pltpu.SemaphoreType.DMA,
            pltpu.SemaphoreType.DMA,
        ],
        compiler_params=pltpu.CompilerParams(use_tc_tiling_on_sc=False),
    )
    def lookup(ids_hbm, tab0_hbm, tab1_hbm, out_hbm, idx_v, ia_v, ib_v,
               rows_v, gsem0, gsem1, osem):
        wid = lax.axis_index("s") * num_cores + lax.axis_index("c")
        base = wid * bpw
        gsems = (gsem0, gsem1)
        pltpu.sync_copy(ids_hbm.at[wid], idx_v)
        ign16 = jnp.full((_NL,), _IGN, jnp.int32)

        def split_s(s, _):
            for g in range(_CHUNK // _NL):
                v16 = idx_v[s, pl.ds(g * _NL, _NL)]
                in0 = v16 < _SPLIT
                ia_v[s, pl.ds(g * _NL, _NL)] = jnp.where(in0, v16, ign16)
                ib_v[s, pl.ds(g * _NL, _NL)] = jnp.where(in0, ign16,
                                                         v16 - _SPLIT)
            return ()

        lax.fori_loop(0, nsup, split_s, (), unroll=False)

        def start_gather(s):
            buf = rows_v.at[s % 2]
            sem = gsems[s % 2]
            return [
                pltpu.async_copy(
                    tab0_hbm.at[plsc.Indices(ia_v.at[s], ignored_value=_IGN)],
                    buf, sem,
                ),
                pltpu.async_copy(
                    tab1_hbm.at[plsc.Indices(ib_v.at[s], ignored_value=_IGN)],
                    buf, sem,
                ),
            ]

        h_g = [None] * nsup
        h_o = [None] * nsup
        h_g[0] = start_gather(0)
        for s in range(nsup):
            if s + 1 < nsup:
                if s >= 1:
                    h_o[s - 1].wait()
                h_g[s + 1] = start_gather(s + 1)
            for h in h_g[s]:
                h.wait()
            h_o[s] = pltpu.async_copy(
                rows_v.at[s % 2], out_hbm.at[pl.ds(base + s * _CHUNK, _CHUNK)], osem
            )
        h_o[nsup - 1].wait()

    return lookup


def kernel(token_ids, embeddings):
    b, t = token_ids.shape
    total = b * t
    info = plsc.get_sparse_core_info()
    nw = info.num_cores * info.num_subcores
    ids = token_ids.reshape(nw, total // (nw * _CHUNK), _CHUNK)
    out = _make_lookup(total, nw, info.num_cores)(
        ids, embeddings[:_SPLIT], embeddings[_SPLIT:]
    )
    return out.reshape(b, t, _D)


# speedup vs baseline: 1.0345x; 1.0345x over previous
import functools

import jax
import jax.numpy as jnp
from jax import lax
from jax.experimental import pallas as pl
from jax.experimental.pallas import tpu as pltpu
from jax.experimental.pallas import tpu_sc as plsc

_D = 32
_CHUNK = 1280


def _make_lookup(total, num_workers, num_cores):
    bpw = total // num_workers
    nsup = bpw // _CHUNK
    mesh = plsc.VectorSubcoreMesh(core_axis_name="c", subcore_axis_name="s")

    @functools.partial(
        pl.kernel,
        out_type=jax.ShapeDtypeStruct((total, _D), jnp.float32),
        mesh=mesh,
        scratch_types=[
            pltpu.VMEM((nsup, _CHUNK), jnp.int32),
            pltpu.VMEM((2, _CHUNK, _D), jnp.float32),
            pltpu.SemaphoreType.DMA,
            pltpu.SemaphoreType.DMA,
            pltpu.SemaphoreType.DMA,
        ],
        compiler_params=pltpu.CompilerParams(use_tc_tiling_on_sc=False),
    )
    def lookup(ids_hbm, table_hbm, out_hbm, idx_v, rows_v, gsem0, gsem1, osem):
        wid = lax.axis_index("s") * num_cores + lax.axis_index("c")
        base = wid * bpw
        gsems = (gsem0, gsem1)
        pltpu.sync_copy(ids_hbm.at[wid], idx_v)

        def start_gather(s):
            return pltpu.async_copy(
                table_hbm.at[idx_v.at[s]], rows_v.at[s % 2], gsems[s % 2]
            )

        h_g = [None] * nsup
        h_o = [None] * nsup
        h_g[0] = start_gather(0)
        for s in range(nsup):
            if s + 1 < nsup:
                if s >= 1:
                    h_o[s - 1].wait()
                h_g[s + 1] = start_gather(s + 1)
            h_g[s].wait()
            h_o[s] = pltpu.async_copy(
                rows_v.at[s % 2], out_hbm.at[pl.ds(base + s * _CHUNK, _CHUNK)], osem
            )
        h_o[nsup - 1].wait()

    return lookup


def kernel(token_ids, embeddings):
    b, t = token_ids.shape
    total = b * t
    info = plsc.get_sparse_core_info()
    nw = info.num_cores * info.num_subcores
    ids = token_ids.reshape(nw, total // (nw * _CHUNK), _CHUNK)
    out = _make_lookup(total, nw, info.num_cores)(ids, embeddings)
    return out.reshape(b, t, _D)
